# baseline (speedup 1.0000x reference)
import functools
import jax
import jax.numpy as jnp
from jax import lax
from jax.experimental import pallas as pl
from jax.experimental.pallas import tpu as pltpu
from jax.experimental.pallas import tpu_sc as plsc

_N = 10000
_H = 128
_E = 320000

_NC = 2
_NS = 16
_NW = _NC * _NS

_CHUNK = 128
_NCH = 80
_EPW = _NCH * _CHUNK
_EP = _EPW * _NW

_NP = 10240
_RPT = _NP // _NS

_BR = 1000


def _sc_mesh():
    return plsc.VectorSubcoreMesh(core_axis_name="c", subcore_axis_name="s")


def _deg_body(dst_hbm, ones_hbm, zeros_hbm, out_hbm, dsts_v, ones_v, acc_sh):
    cid = lax.axis_index("c")
    sid = lax.axis_index("s")
    wid = cid * _NS + sid
    pltpu.sync_copy(ones_hbm, ones_v)
    pltpu.sync_copy(zeros_hbm.at[pl.ds(sid * _RPT, _RPT)],
                    acc_sh.at[pl.ds(sid * _RPT, _RPT)])
    plsc.subcore_barrier()

    def body(k, carry):
        pltpu.sync_copy(dst_hbm.at[wid, k], dsts_v)
        pltpu.sync_copy(ones_v, acc_sh.at[dsts_v], add=True)
        return carry

    lax.fori_loop(0, _NCH, body, 0)
    plsc.subcore_barrier()
    pltpu.sync_copy(acc_sh.at[pl.ds(sid * _RPT, _RPT)],
                    out_hbm.at[cid, pl.ds(sid * _RPT, _RPT)])


def _deg_partials(dst3, ones_c, zeros_1d):
    return pl.kernel(
        _deg_body,
        out_type=jax.ShapeDtypeStruct((_NC, _NP), jnp.float32),
        mesh=_sc_mesh(),
        scratch_types=[
            pltpu.VMEM((_CHUNK,), jnp.int32),
            pltpu.VMEM((_CHUNK,), jnp.float32),
            pltpu.VMEM_SHARED((_NP,), jnp.float32),
        ],
    )(dst3, ones_c, zeros_1d)


def _edges_body(hp_hbm, src_hbm, dst_hbm, zeros_hbm, out_hbm,
                s0, s1, d0, d1, r0, r1, acc_sh, gs0, gs1):
    cid = lax.axis_index("c")
    sid = lax.axis_index("s")
    wid = cid * _NS + sid
    src_v = (s0, s1)
    dst_v = (d0, d1)
    rows = (r0, r1)
    gsem = (gs0, gs1)
    pltpu.sync_copy(zeros_hbm.at[pl.ds(sid * _RPT, _RPT)],
                    acc_sh.at[pl.ds(sid * _RPT, _RPT)])
    pltpu.sync_copy(src_hbm.at[pl.ds(wid * _EPW, _CHUNK)], s0)
    pltpu.sync_copy(dst_hbm.at[pl.ds(wid * _EPW, _CHUNK)], d0)
    plsc.subcore_barrier()
    pltpu.async_copy(hp_hbm.at[s0], r0, gs0)

    def body2(k2, carry):
        for b in range(2):
            k = k2 * 2 + b
            bn = 1 - b
            pltpu.make_async_copy(hp_hbm.at[pl.ds(0, _CHUNK)],
                                  rows[b], gsem[b]).wait()

            @pl.when(k + 1 < _NCH)
            def _():
                base_n = wid * _EPW + (k + 1) * _CHUNK
                pltpu.sync_copy(src_hbm.at[pl.ds(base_n, _CHUNK)], src_v[bn])
                pltpu.sync_copy(dst_hbm.at[pl.ds(base_n, _CHUNK)], dst_v[bn])
                pltpu.async_copy(hp_hbm.at[src_v[bn]], rows[bn], gsem[bn])

            pltpu.sync_copy(rows[b], acc_sh.at[dst_v[b]], add=True)
        return carry

    lax.fori_loop(0, _NCH // 2, body2, 0)
    plsc.subcore_barrier()
    pltpu.sync_copy(acc_sh.at[pl.ds(sid * _RPT, _RPT)],
                    out_hbm.at[cid, pl.ds(sid * _RPT, _RPT)])


def _edge_scatter(hp, srcp, dstp, zeros_2d):
    return pl.kernel(
        _edges_body,
        out_type=jax.ShapeDtypeStruct((_NC, _NP, _H), jnp.float32),
        mesh=_sc_mesh(),
        scratch_types=[
            pltpu.VMEM((_CHUNK,), jnp.int32),
            pltpu.VMEM((_CHUNK,), jnp.int32),
            pltpu.VMEM((_CHUNK,), jnp.int32),
            pltpu.VMEM((_CHUNK,), jnp.int32),
            pltpu.VMEM((_CHUNK, _H), jnp.float32),
            pltpu.VMEM((_CHUNK, _H), jnp.float32),
            pltpu.VMEM_SHARED((_NP, _H), jnp.float32),
            pltpu.SemaphoreType.DMA,
            pltpu.SemaphoreType.DMA,
        ],
    )(hp, srcp, dstp, zeros_2d)


def _hp_body(d0_ref, d1_ref, *refs):
    nx = (len(refs) - 1) // 2
    x_refs = refs[:nx]
    w_refs = refs[nx:2 * nx]
    out_ref = refs[2 * nx]
    h = jnp.dot(x_refs[0][...], w_refs[0][...], preferred_element_type=jnp.float32)
    for j in range(1, nx):
        h += jnp.dot(x_refs[j][...], w_refs[j][...], preferred_element_type=jnp.float32)
    dinv = lax.rsqrt(1.0 + d0_ref[...] + d1_ref[...])
    out_ref[...] = dinv * h


def _hprime(x_blocks, W, deg0, deg1):
    nx = len(x_blocks)
    w_parts = [W[j * _H:(j + 1) * _H] for j in range(nx)]
    grid = _N // _BR
    in_specs = (
        [pl.BlockSpec((_BR, 1), lambda i: (i, 0)),
         pl.BlockSpec((_BR, 1), lambda i: (i, 0))]
        + [pl.BlockSpec((_BR, _H), lambda i: (i, 0))] * nx
        + [pl.BlockSpec((_H, _H), lambda i: (0, 0))] * nx
    )
    return pl.pallas_call(
        _hp_body,
        grid=(grid,),
        in_specs=in_specs,
        out_specs=pl.BlockSpec((_BR, _H), lambda i: (i, 0)),
        out_shape=jax.ShapeDtypeStruct((_N, _H), jnp.float32),
    )(deg0, deg1, *x_blocks, *w_parts)


def _epi_body(xp_ref, s0_ref, s1_ref, hp_ref, d0_ref, d1_ref, b_ref, out_ref):
    dinv = lax.rsqrt(1.0 + d0_ref[...] + d1_ref[...])
    agg = dinv * (s0_ref[0] + s1_ref[0] + hp_ref[...]) + b_ref[...]
    out_ref[...] = jnp.maximum(xp_ref[...] + agg, 0.0)


def _epilogue(x_prev, parts, hp, deg0, deg1, b):
    grid = _N // _BR
    return pl.pallas_call(
        _epi_body,
        grid=(grid,),
        in_specs=[
            pl.BlockSpec((_BR, _H), lambda i: (i, 0)),
            pl.BlockSpec((1, _BR, _H), lambda i: (0, i, 0)),
            pl.BlockSpec((1, _BR, _H), lambda i: (1, i, 0)),
            pl.BlockSpec((_BR, _H), lambda i: (i, 0)),
            pl.BlockSpec((_BR, 1), lambda i: (i, 0)),
            pl.BlockSpec((_BR, 1), lambda i: (i, 0)),
            pl.BlockSpec((1, _H), lambda i: (0, 0)),
        ],
        out_specs=pl.BlockSpec((_BR, _H), lambda i: (i, 0)),
        out_shape=jax.ShapeDtypeStruct((_N, _H), jnp.float32),
    )(x_prev, parts, parts, hp, deg0, deg1, b.reshape(1, _H))


def kernel(x, edge_index, W1, b1, W2, b2, W3, b3, W4, b4, W5, b5):
    src = edge_index[0]
    dst = edge_index[1]
    pad = _EP - _E
    ppos = _E + jnp.arange(pad, dtype=jnp.int32)
    dump = _N + (ppos // _CHUNK) % _NW
    srcp = jnp.concatenate([src, jnp.zeros((pad,), jnp.int32)])
    dstp = jnp.concatenate([dst, dump])
    srcp = srcp.reshape(_NCH, _NW, _CHUNK).transpose(1, 0, 2).reshape(-1)
    dstp = dstp.reshape(_NCH, _NW, _CHUNK).transpose(1, 0, 2).reshape(-1)
    dst3 = dstp.reshape(_NW, _NCH, _CHUNK)
    ones_c = jnp.ones((_CHUNK,), jnp.float32)
    zeros_1d = jnp.zeros((_NP,), jnp.float32)
    zeros_2d = jnp.zeros((_NP, _H), jnp.float32)

    degs = _deg_partials(dst3, ones_c, zeros_1d)
    degs3 = degs.reshape(_NC, _NP, 1)
    deg0 = degs3[0]
    deg1 = degs3[1]

    Ws = [W1, W2, W3, W4, W5]
    bs = [b1, b2, b3, b4, b5]
    blocks = [x]
    x_prev = x
    for W, b in zip(Ws, bs):
        hp = _hprime(blocks, W, deg0, deg1)
        parts = _edge_scatter(hp, srcp, dstp, zeros_2d)
        x_new = _epilogue(x_prev, parts, hp, deg0, deg1, b)
        blocks.append(x_new)
        x_prev = x_new
    return jnp.concatenate(blocks, axis=1)


# speedup vs baseline: 1.2602x; 1.2602x over previous
import functools
import jax
import jax.numpy as jnp
from jax import lax
from jax.experimental import pallas as pl
from jax.experimental.pallas import tpu as pltpu
from jax.experimental.pallas import tpu_sc as plsc

_N = 10000
_H = 128
_E = 320000

_NC = 2
_NS = 16
_NW = _NC * _NS

_CHUNK = 128
_NCH = 79
_EPW = _NCH * _CHUNK
_EP = _EPW * _NW

_NP = 10240
_RPT = _NP // _NS

_BR = 1000


def _sc_mesh():
    return plsc.VectorSubcoreMesh(core_axis_name="c", subcore_axis_name="s")


def _deg_body(dst_hbm, ones_hbm, zeros_hbm, out_hbm, dsts_v, ones_v, acc_sh):
    cid = lax.axis_index("c")
    sid = lax.axis_index("s")
    wid = cid * _NS + sid
    pltpu.sync_copy(ones_hbm, ones_v)
    pltpu.sync_copy(zeros_hbm.at[pl.ds(sid * _RPT, _RPT)],
                    acc_sh.at[pl.ds(sid * _RPT, _RPT)])
    plsc.subcore_barrier()

    def body(k, carry):
        pltpu.sync_copy(dst_hbm.at[wid, k], dsts_v)
        pltpu.sync_copy(ones_v, acc_sh.at[dsts_v], add=True)
        return carry

    lax.fori_loop(0, _NCH, body, 0)
    plsc.subcore_barrier()
    pltpu.sync_copy(acc_sh.at[pl.ds(sid * _RPT, _RPT)],
                    out_hbm.at[cid, pl.ds(sid * _RPT, _RPT)])


def _deg_partials(dst3, ones_c, zeros_1d):
    return pl.kernel(
        _deg_body,
        out_type=jax.ShapeDtypeStruct((_NC, _NP), jnp.float32),
        mesh=_sc_mesh(),
        scratch_types=[
            pltpu.VMEM((_CHUNK,), jnp.int32),
            pltpu.VMEM((_CHUNK,), jnp.float32),
            pltpu.VMEM_SHARED((_NP,), jnp.float32),
        ],
    )(dst3, ones_c, zeros_1d)


def _edges_body(hp_hbm, src_hbm, dst_hbm, zeros_hbm, out_hbm,
                src_v, dst_v, rows_v, acc_sh, sem):
    cid = lax.axis_index("c")
    sid = lax.axis_index("s")
    wid = cid * _NS + sid
    pltpu.sync_copy(zeros_hbm.at[pl.ds(sid * _RPT, _RPT)],
                    acc_sh.at[pl.ds(sid * _RPT, _RPT)])
    plsc.subcore_barrier()

    def body(k, carry):
        base = wid * _EPW + k * _CHUNK
        pltpu.sync_copy(src_hbm.at[pl.ds(base, _CHUNK)], src_v)
        pltpu.sync_copy(dst_hbm.at[pl.ds(base, _CHUNK)], dst_v)
        pltpu.async_copy(hp_hbm.at[src_v], rows_v, sem).wait()
        pltpu.sync_copy(rows_v, acc_sh.at[dst_v], add=True)
        return carry

    lax.fori_loop(0, _NCH, body, 0)
    plsc.subcore_barrier()
    pltpu.sync_copy(acc_sh.at[pl.ds(sid * _RPT, _RPT)],
                    out_hbm.at[cid, pl.ds(sid * _RPT, _RPT)])


def _edge_scatter(hp, srcp, dstp, zeros_2d):
    return pl.kernel(
        _edges_body,
        out_type=jax.ShapeDtypeStruct((_NC, _NP, _H), jnp.float32),
        mesh=_sc_mesh(),
        scratch_types=[
            pltpu.VMEM((_CHUNK,), jnp.int32),
            pltpu.VMEM((_CHUNK,), jnp.int32),
            pltpu.VMEM((_CHUNK, _H), jnp.float32),
            pltpu.VMEM_SHARED((_NP, _H), jnp.float32),
            pltpu.SemaphoreType.DMA,
        ],
    )(hp, srcp, dstp, zeros_2d)


def _hp_body(d0_ref, d1_ref, *refs):
    nx = (len(refs) - 1) // 2
    x_refs = refs[:nx]
    w_refs = refs[nx:2 * nx]
    out_ref = refs[2 * nx]
    h = jnp.dot(x_refs[0][...], w_refs[0][...], preferred_element_type=jnp.float32)
    for j in range(1, nx):
        h += jnp.dot(x_refs[j][...], w_refs[j][...], preferred_element_type=jnp.float32)
    dinv = lax.rsqrt(1.0 + d0_ref[...] + d1_ref[...])
    out_ref[...] = dinv * h


def _hprime(x_blocks, W, deg0, deg1):
    nx = len(x_blocks)
    w_parts = [W[j * _H:(j + 1) * _H] for j in range(nx)]
    grid = _N // _BR
    in_specs = (
        [pl.BlockSpec((_BR, 1), lambda i: (i, 0)),
         pl.BlockSpec((_BR, 1), lambda i: (i, 0))]
        + [pl.BlockSpec((_BR, _H), lambda i: (i, 0))] * nx
        + [pl.BlockSpec((_H, _H), lambda i: (0, 0))] * nx
    )
    return pl.pallas_call(
        _hp_body,
        grid=(grid,),
        in_specs=in_specs,
        out_specs=pl.BlockSpec((_BR, _H), lambda i: (i, 0)),
        out_shape=jax.ShapeDtypeStruct((_N, _H), jnp.float32),
    )(deg0, deg1, *x_blocks, *w_parts)


def _epi_body(xp_ref, s0_ref, s1_ref, hp_ref, d0_ref, d1_ref, b_ref, out_ref):
    dinv = lax.rsqrt(1.0 + d0_ref[...] + d1_ref[...])
    agg = dinv * (s0_ref[0] + s1_ref[0] + hp_ref[...]) + b_ref[...]
    out_ref[...] = jnp.maximum(xp_ref[...] + agg, 0.0)


def _epilogue(x_prev, parts, hp, deg0, deg1, b):
    grid = _N // _BR
    return pl.pallas_call(
        _epi_body,
        grid=(grid,),
        in_specs=[
            pl.BlockSpec((_BR, _H), lambda i: (i, 0)),
            pl.BlockSpec((1, _BR, _H), lambda i: (0, i, 0)),
            pl.BlockSpec((1, _BR, _H), lambda i: (1, i, 0)),
            pl.BlockSpec((_BR, _H), lambda i: (i, 0)),
            pl.BlockSpec((_BR, 1), lambda i: (i, 0)),
            pl.BlockSpec((_BR, 1), lambda i: (i, 0)),
            pl.BlockSpec((1, _H), lambda i: (0, 0)),
        ],
        out_specs=pl.BlockSpec((_BR, _H), lambda i: (i, 0)),
        out_shape=jax.ShapeDtypeStruct((_N, _H), jnp.float32),
    )(x_prev, parts, parts, hp, deg0, deg1, b.reshape(1, _H))


def kernel(x, edge_index, W1, b1, W2, b2, W3, b3, W4, b4, W5, b5):
    src = edge_index[0]
    dst = edge_index[1]
    pad = _EP - _E
    ppos = _E + jnp.arange(pad, dtype=jnp.int32)
    dump = _N + (ppos // _CHUNK) % _NW
    srcp = jnp.concatenate([src, jnp.zeros((pad,), jnp.int32)])
    dstp = jnp.concatenate([dst, dump])
    srcp = srcp.reshape(_NCH, _NW, _CHUNK).transpose(1, 0, 2).reshape(-1)
    dstp = dstp.reshape(_NCH, _NW, _CHUNK).transpose(1, 0, 2).reshape(-1)
    dst3 = dstp.reshape(_NW, _NCH, _CHUNK)
    ones_c = jnp.ones((_CHUNK,), jnp.float32)
    zeros_1d = jnp.zeros((_NP,), jnp.float32)
    zeros_2d = jnp.zeros((_NP, _H), jnp.float32)

    degs = _deg_partials(dst3, ones_c, zeros_1d)
    degs3 = degs.reshape(_NC, _NP, 1)
    deg0 = degs3[0]
    deg1 = degs3[1]

    Ws = [W1, W2, W3, W4, W5]
    bs = [b1, b2, b3, b4, b5]
    blocks = [x]
    x_prev = x
    for W, b in zip(Ws, bs):
        hp = _hprime(blocks, W, deg0, deg1)
        parts = _edge_scatter(hp, srcp, dstp, zeros_2d)
        x_new = _epilogue(x_prev, parts, hp, deg0, deg1, b)
        blocks.append(x_new)
        x_prev = x_new
    return jnp.concatenate(blocks, axis=1)
